# Initial kernel scaffold; baseline (speedup 1.0000x reference)
import jax
import jax.numpy as jnp
from jax.experimental import pallas as pl


def kernel(q3d, k3d, v3d, values, row_indices, row_offsets, column_indices):
    raise NotImplementedError("write your pallas kernel here")


# speedup vs baseline: 6.2795x; 6.2795x over previous
import functools

import jax
import jax.numpy as jnp
from jax import lax
from jax.experimental import pallas as pl
from jax.experimental.pallas import tpu as pltpu
from jax.experimental.pallas import tpu_sc as plsc

RHEADS = 12
MROWS = 2048
NCOLS = 2048
DHEAD = 64
PER_ROW = 128

NCORES = 2
NSUBCORES = 16
NWORKERS = NCORES * NSUBCORES
ROWS_PER_W = MROWS // NWORKERS
CHUNK_ROWS = 32
NCHUNKS = ROWS_PER_W // CHUNK_ROWS
VECS_PER_ROW = PER_ROW // 16


def _sc_build_mult_body(cols_hbm, zeros_hbm, mult_hbm, idx_v, buf_v):
    wid = lax.axis_index("s") * NCORES + lax.axis_index("c")
    ones16 = jnp.ones((16,), jnp.float32)
    for ci in range(NCHUNKS):
        r0 = wid * ROWS_PER_W + ci * CHUNK_ROWS
        pltpu.sync_copy(cols_hbm.at[pl.ds(r0 * PER_ROW, CHUNK_ROWS * PER_ROW)],
                        idx_v)
        pltpu.sync_copy(zeros_hbm, buf_v)

        def vec_body(i, carry):
            idx = idx_v[pl.ds(i * 16, 16)]
            flat = idx + (i // VECS_PER_ROW) * NCOLS
            plsc.addupdate_scatter(buf_v, [flat], ones16)
            return carry

        lax.fori_loop(0, CHUNK_ROWS * VECS_PER_ROW, vec_body, 0)
        pltpu.sync_copy(buf_v,
                        mult_hbm.at[pl.ds(r0 * NCOLS, CHUNK_ROWS * NCOLS)])


def _build_mult(column_indices):
    mesh = plsc.VectorSubcoreMesh(core_axis_name="c", subcore_axis_name="s")
    zeros = jnp.zeros((CHUNK_ROWS * NCOLS,), jnp.float32)
    fn = functools.partial(
        pl.kernel,
        out_type=jax.ShapeDtypeStruct((MROWS * NCOLS,), jnp.float32),
        mesh=mesh,
        scratch_types=[
            pltpu.VMEM((CHUNK_ROWS * PER_ROW,), jnp.int32),
            pltpu.VMEM((CHUNK_ROWS * NCOLS,), jnp.float32),
        ],
        compiler_params=pltpu.CompilerParams(needs_layout_passes=False),
    )(_sc_build_mult_body)
    return fn(column_indices, zeros).reshape(MROWS, NCOLS)


BM = 256


def _tc_attn_body(q_ref, k_ref, v_ref, mult_ref, o_ref):
    q = q_ref[0]
    k = k_ref[0]
    v = v_ref[0]
    mult = mult_ref[...]
    s = lax.dot_general(q, k, (((1,), (1,)), ((), ())),
                        precision=lax.Precision.HIGHEST,
                        preferred_element_type=jnp.float32)
    sm = jnp.where(mult > 0.0, s, -1e30)
    rowmax = jnp.max(sm, axis=1, keepdims=True)
    e = mult * jnp.exp(sm - rowmax)
    z = jnp.sum(e, axis=1, keepdims=True)
    o = lax.dot_general(e, v, (((1,), (0,)), ((), ())),
                        precision=lax.Precision.HIGHEST,
                        preferred_element_type=jnp.float32)
    o_ref[0] = o / z


def _attention_tc(q3d, k3d, v3d, mult, interpret=False):
    grid = (MROWS // BM, RHEADS)
    return pl.pallas_call(
        _tc_attn_body,
        grid=grid,
        in_specs=[
            pl.BlockSpec((1, BM, DHEAD), lambda i, r: (r, i, 0)),
            pl.BlockSpec((1, NCOLS, DHEAD), lambda i, r: (r, 0, 0)),
            pl.BlockSpec((1, NCOLS, DHEAD), lambda i, r: (r, 0, 0)),
            pl.BlockSpec((BM, NCOLS), lambda i, r: (i, 0)),
        ],
        out_specs=pl.BlockSpec((1, BM, DHEAD), lambda i, r: (r, i, 0)),
        out_shape=jax.ShapeDtypeStruct((RHEADS, MROWS, DHEAD), jnp.float32),
        interpret=interpret,
    )(q3d, k3d, v3d, mult)


def kernel(q3d, k3d, v3d, values, row_indices, row_offsets, column_indices):
    mult = _build_mult(column_indices)
    return _attention_tc(q3d, k3d, v3d, mult)


# speedup vs baseline: 12.6378x; 2.0125x over previous
import functools

import jax
import jax.numpy as jnp
from jax import lax
from jax.experimental import pallas as pl
from jax.experimental.pallas import tpu as pltpu
from jax.experimental.pallas import tpu_sc as plsc

RHEADS = 12
MROWS = 2048
NCOLS = 2048
DHEAD = 64
PER_ROW = 128

NCORES = 2
NSUBCORES = 16
NWORKERS = NCORES * NSUBCORES
ROWS_PER_W = MROWS // NWORKERS
CHUNK_ROWS = 32
NCHUNKS = ROWS_PER_W // CHUNK_ROWS
VECS_PER_ROW = PER_ROW // 16


def _sc_build_mult_body(cols_hbm, zeros_hbm, mult_hbm, idx_v, buf_v):
    wid = lax.axis_index("s") * NCORES + lax.axis_index("c")
    ones16 = jnp.ones((16,), jnp.float32)
    for ci in range(NCHUNKS):
        r0 = wid * ROWS_PER_W + ci * CHUNK_ROWS
        pltpu.sync_copy(cols_hbm.at[pl.ds(r0 * PER_ROW, CHUNK_ROWS * PER_ROW)],
                        idx_v)
        pltpu.sync_copy(zeros_hbm, buf_v)

        def vec_body(i, carry):
            idx = idx_v[pl.ds(i * 16, 16)]
            flat = idx + (i // VECS_PER_ROW) * NCOLS
            plsc.addupdate_scatter(buf_v, [flat], ones16)
            return carry

        lax.fori_loop(0, CHUNK_ROWS * VECS_PER_ROW, vec_body, 0)
        pltpu.sync_copy(buf_v,
                        mult_hbm.at[pl.ds(r0 * NCOLS, CHUNK_ROWS * NCOLS)])


def _build_mult(column_indices):
    mesh = plsc.VectorSubcoreMesh(core_axis_name="c", subcore_axis_name="s")
    zeros = jnp.zeros((CHUNK_ROWS * NCOLS,), jnp.float32)
    fn = functools.partial(
        pl.kernel,
        out_type=jax.ShapeDtypeStruct((MROWS * NCOLS,), jnp.float32),
        mesh=mesh,
        scratch_types=[
            pltpu.VMEM((CHUNK_ROWS * PER_ROW,), jnp.int32),
            pltpu.VMEM((CHUNK_ROWS * NCOLS,), jnp.float32),
        ],
        compiler_params=pltpu.CompilerParams(needs_layout_passes=False),
    )(_sc_build_mult_body)
    return fn(column_indices, zeros).reshape(MROWS, NCOLS)


BM = 256


def _tc_attn_body(q_ref, k_ref, v_ref, mult_ref, o_ref):
    q = q_ref[0]
    k = k_ref[0]
    v = v_ref[0]
    mult = mult_ref[...]
    s = lax.dot_general(q, k, (((1,), (1,)), ((), ())),
                        precision=lax.Precision.HIGHEST,
                        preferred_element_type=jnp.float32)
    sm = jnp.where(mult > 0.0, s, -1e30)
    rowmax = jnp.max(sm, axis=1, keepdims=True)
    e = mult * jnp.exp(sm - rowmax)
    z = jnp.sum(e, axis=1, keepdims=True)
    o = lax.dot_general(e, v, (((1,), (0,)), ((), ())),
                        precision=lax.Precision.DEFAULT,
                        preferred_element_type=jnp.float32)
    o_ref[0] = o / z


def _attention_tc(q3d, k3d, v3d, mult, interpret=False):
    grid = (MROWS // BM, RHEADS)
    return pl.pallas_call(
        _tc_attn_body,
        grid=grid,
        in_specs=[
            pl.BlockSpec((1, BM, DHEAD), lambda i, r: (r, i, 0)),
            pl.BlockSpec((1, NCOLS, DHEAD), lambda i, r: (r, 0, 0)),
            pl.BlockSpec((1, NCOLS, DHEAD), lambda i, r: (r, 0, 0)),
            pl.BlockSpec((BM, NCOLS), lambda i, r: (i, 0)),
        ],
        out_specs=pl.BlockSpec((1, BM, DHEAD), lambda i, r: (r, i, 0)),
        out_shape=jax.ShapeDtypeStruct((RHEADS, MROWS, DHEAD), jnp.float32),
        interpret=interpret,
    )(q3d, k3d, v3d, mult)


def kernel(q3d, k3d, v3d, values, row_indices, row_offsets, column_indices):
    mult = _build_mult(column_indices)
    return _attention_tc(q3d, k3d, v3d, mult)


# speedup vs baseline: 15.9084x; 1.2588x over previous
import functools

import jax
import jax.numpy as jnp
from jax import lax
from jax.experimental import pallas as pl
from jax.experimental.pallas import tpu as pltpu
from jax.experimental.pallas import tpu_sc as plsc

RHEADS = 12
MROWS = 2048
NCOLS = 2048
DHEAD = 64
PER_ROW = 128

NCORES = 2
NSUBCORES = 16
NWORKERS = NCORES * NSUBCORES
ROWS_PER_W = MROWS // NWORKERS
CHUNK_ROWS = 32
NCHUNKS = ROWS_PER_W // CHUNK_ROWS
VECS_PER_ROW = PER_ROW // 16


def _sc_build_mult_body(cols_hbm, zeros_hbm, mult_hbm, idx_v, buf_v):
    wid = lax.axis_index("s") * NCORES + lax.axis_index("c")
    ones16 = jnp.ones((16,), jnp.float32)
    for ci in range(NCHUNKS):
        r0 = wid * ROWS_PER_W + ci * CHUNK_ROWS
        pltpu.sync_copy(cols_hbm.at[pl.ds(r0 * PER_ROW, CHUNK_ROWS * PER_ROW)],
                        idx_v)
        pltpu.sync_copy(zeros_hbm, buf_v)

        def vec_body(i, carry):
            idx = idx_v[pl.ds(i * 16, 16)]
            flat = idx + (i // VECS_PER_ROW) * NCOLS
            plsc.addupdate_scatter(buf_v, [flat], ones16)
            return carry

        lax.fori_loop(0, CHUNK_ROWS * VECS_PER_ROW, vec_body, 0)
        pltpu.sync_copy(buf_v,
                        mult_hbm.at[pl.ds(r0 * NCOLS, CHUNK_ROWS * NCOLS)])


def _build_mult(column_indices):
    mesh = plsc.VectorSubcoreMesh(core_axis_name="c", subcore_axis_name="s")
    zeros = jnp.zeros((CHUNK_ROWS * NCOLS,), jnp.float32)
    fn = functools.partial(
        pl.kernel,
        out_type=jax.ShapeDtypeStruct((MROWS * NCOLS,), jnp.float32),
        mesh=mesh,
        scratch_types=[
            pltpu.VMEM((CHUNK_ROWS * PER_ROW,), jnp.int32),
            pltpu.VMEM((CHUNK_ROWS * NCOLS,), jnp.float32),
        ],
        compiler_params=pltpu.CompilerParams(needs_layout_passes=False),
    )(_sc_build_mult_body)
    return fn(column_indices, zeros).reshape(MROWS, NCOLS)


BM = 256


def _tc_attn_body(q_ref, k_ref, v_ref, mult_ref, o_ref):
    q = q_ref[0]
    k = k_ref[0]
    v = v_ref[0]
    mult = mult_ref[...]
    s = lax.dot_general(q, k, (((1,), (1,)), ((), ())),
                        precision=lax.Precision.HIGHEST,
                        preferred_element_type=jnp.float32)
    e = mult * jnp.exp(jnp.minimum(s, 60.0))
    z = jnp.sum(e, axis=1, keepdims=True)
    o = lax.dot_general(e, v, (((1,), (0,)), ((), ())),
                        precision=lax.Precision.DEFAULT,
                        preferred_element_type=jnp.float32)
    o_ref[0] = o / z


def _attention_tc(q3d, k3d, v3d, mult, interpret=False):
    grid = (MROWS // BM, RHEADS)
    return pl.pallas_call(
        _tc_attn_body,
        grid=grid,
        in_specs=[
            pl.BlockSpec((1, BM, DHEAD), lambda i, r: (r, i, 0)),
            pl.BlockSpec((1, NCOLS, DHEAD), lambda i, r: (r, 0, 0)),
            pl.BlockSpec((1, NCOLS, DHEAD), lambda i, r: (r, 0, 0)),
            pl.BlockSpec((BM, NCOLS), lambda i, r: (i, 0)),
        ],
        out_specs=pl.BlockSpec((1, BM, DHEAD), lambda i, r: (r, i, 0)),
        out_shape=jax.ShapeDtypeStruct((RHEADS, MROWS, DHEAD), jnp.float32),
        interpret=interpret,
    )(q3d, k3d, v3d, mult)


def kernel(q3d, k3d, v3d, values, row_indices, row_offsets, column_indices):
    mult = _build_mult(column_indices)
    return _attention_tc(q3d, k3d, v3d, mult)


# speedup vs baseline: 18.9368x; 1.1904x over previous
import functools

import jax
import jax.numpy as jnp
from jax import lax
from jax.experimental import pallas as pl
from jax.experimental.pallas import tpu as pltpu
from jax.experimental.pallas import tpu_sc as plsc

RHEADS = 12
MROWS = 2048
NCOLS = 2048
DHEAD = 64
PER_ROW = 128

NCORES = 2
NSUBCORES = 16
NWORKERS = NCORES * NSUBCORES
ROWS_PER_W = MROWS // NWORKERS
CHUNK_ROWS = 32
NCHUNKS = ROWS_PER_W // CHUNK_ROWS
VECS_PER_ROW = PER_ROW // 16


def _sc_build_mult_body(cols_hbm, zeros_hbm, mult_hbm, idx_v, buf_v):
    wid = lax.axis_index("s") * NCORES + lax.axis_index("c")
    ones16 = jnp.ones((16,), jnp.float32)
    for ci in range(NCHUNKS):
        r0 = wid * ROWS_PER_W + ci * CHUNK_ROWS
        pltpu.sync_copy(cols_hbm.at[pl.ds(r0 * PER_ROW, CHUNK_ROWS * PER_ROW)],
                        idx_v)
        pltpu.sync_copy(zeros_hbm, buf_v)

        def vec_body(i, carry):
            idx = idx_v[pl.ds(i * 16, 16)]
            flat = idx + (i // VECS_PER_ROW) * NCOLS
            plsc.addupdate_scatter(buf_v, [flat], ones16)
            return carry

        lax.fori_loop(0, CHUNK_ROWS * VECS_PER_ROW, vec_body, 0)
        pltpu.sync_copy(buf_v,
                        mult_hbm.at[pl.ds(r0 * NCOLS, CHUNK_ROWS * NCOLS)])


def _build_mult(column_indices):
    mesh = plsc.VectorSubcoreMesh(core_axis_name="c", subcore_axis_name="s")
    zeros = jnp.zeros((CHUNK_ROWS * NCOLS,), jnp.float32)
    fn = functools.partial(
        pl.kernel,
        out_type=jax.ShapeDtypeStruct((MROWS * NCOLS,), jnp.float32),
        mesh=mesh,
        scratch_types=[
            pltpu.VMEM((CHUNK_ROWS * PER_ROW,), jnp.int32),
            pltpu.VMEM((CHUNK_ROWS * NCOLS,), jnp.float32),
        ],
        compiler_params=pltpu.CompilerParams(needs_layout_passes=False),
    )(_sc_build_mult_body)
    return fn(column_indices, zeros).reshape(MROWS, NCOLS)


BM = 256


def _tc_attn_body(q_ref, k_ref, v_ref, mult_ref, o_ref):
    q = q_ref[0]
    k = k_ref[0]
    v = v_ref[0]
    mult = mult_ref[...]
    q_hi = q.astype(jnp.bfloat16)
    k_hi = k.astype(jnp.bfloat16)
    q_lo = (q - q_hi.astype(jnp.float32)).astype(jnp.bfloat16)
    k_lo = (k - k_hi.astype(jnp.float32)).astype(jnp.bfloat16)
    dims = (((1,), (1,)), ((), ()))
    s = lax.dot_general(q_hi, k_hi, dims,
                        preferred_element_type=jnp.float32)
    s = s + lax.dot_general(q_hi, k_lo, dims,
                            preferred_element_type=jnp.float32)
    s = s + lax.dot_general(q_lo, k_hi, dims,
                            preferred_element_type=jnp.float32)
    e = mult * jnp.exp(jnp.minimum(s, 60.0))
    z = jnp.sum(e, axis=1, keepdims=True)
    o = lax.dot_general(e, v, (((1,), (0,)), ((), ())),
                        precision=lax.Precision.DEFAULT,
                        preferred_element_type=jnp.float32)
    o_ref[0] = o / z


def _attention_tc(q3d, k3d, v3d, mult, interpret=False):
    grid = (MROWS // BM, RHEADS)
    return pl.pallas_call(
        _tc_attn_body,
        grid=grid,
        in_specs=[
            pl.BlockSpec((1, BM, DHEAD), lambda i, r: (r, i, 0)),
            pl.BlockSpec((1, NCOLS, DHEAD), lambda i, r: (r, 0, 0)),
            pl.BlockSpec((1, NCOLS, DHEAD), lambda i, r: (r, 0, 0)),
            pl.BlockSpec((BM, NCOLS), lambda i, r: (i, 0)),
        ],
        out_specs=pl.BlockSpec((1, BM, DHEAD), lambda i, r: (r, i, 0)),
        out_shape=jax.ShapeDtypeStruct((RHEADS, MROWS, DHEAD), jnp.float32),
        interpret=interpret,
    )(q3d, k3d, v3d, mult)


def kernel(q3d, k3d, v3d, values, row_indices, row_offsets, column_indices):
    mult = _build_mult(column_indices)
    return _attention_tc(q3d, k3d, v3d, mult)


# speedup vs baseline: 19.0637x; 1.0067x over previous
import functools

import jax
import jax.numpy as jnp
from jax import lax
from jax.experimental import pallas as pl
from jax.experimental.pallas import tpu as pltpu
from jax.experimental.pallas import tpu_sc as plsc

RHEADS = 12
MROWS = 2048
NCOLS = 2048
DHEAD = 64
PER_ROW = 128

LOG2E = 1.4426950408889634
DCAT = 3 * DHEAD
DV = DHEAD + 1

NCORES = 2
NSUBCORES = 16
NWORKERS = NCORES * NSUBCORES
ROWS_PER_W = MROWS // NWORKERS
CHUNK_ROWS = 32
NCHUNKS = ROWS_PER_W // CHUNK_ROWS
VECS_PER_ROW = PER_ROW // 16


def _sc_build_mult_body(cols_hbm, zeros_hbm, mult_hbm, idx_v, buf_v):
    wid = lax.axis_index("s") * NCORES + lax.axis_index("c")
    ones16 = jnp.ones((16,), jnp.float32)
    for ci in range(NCHUNKS):
        r0 = wid * ROWS_PER_W + ci * CHUNK_ROWS
        pltpu.sync_copy(cols_hbm.at[pl.ds(r0 * PER_ROW, CHUNK_ROWS * PER_ROW)],
                        idx_v)
        pltpu.sync_copy(zeros_hbm, buf_v)

        def vec_body(i, carry):
            idx = idx_v[pl.ds(i * 16, 16)]
            flat = idx + (i // VECS_PER_ROW) * NCOLS
            plsc.addupdate_scatter(buf_v, [flat], ones16)
            return carry

        lax.fori_loop(0, CHUNK_ROWS * VECS_PER_ROW, vec_body, 0)
        pltpu.sync_copy(buf_v,
                        mult_hbm.at[pl.ds(r0 * NCOLS, CHUNK_ROWS * NCOLS)])


def _build_mult(column_indices):
    mesh = plsc.VectorSubcoreMesh(core_axis_name="c", subcore_axis_name="s")
    zeros = jnp.zeros((CHUNK_ROWS * NCOLS,), jnp.float32)
    fn = functools.partial(
        pl.kernel,
        out_type=jax.ShapeDtypeStruct((MROWS * NCOLS,), jnp.float32),
        mesh=mesh,
        scratch_types=[
            pltpu.VMEM((CHUNK_ROWS * PER_ROW,), jnp.int32),
            pltpu.VMEM((CHUNK_ROWS * NCOLS,), jnp.float32),
        ],
        compiler_params=pltpu.CompilerParams(needs_layout_passes=False),
    )(_sc_build_mult_body)
    return fn(column_indices, zeros).reshape(MROWS, NCOLS)


BM = 256


def _tc_attn_body(q_ref, k_ref, v_ref, mult_ref, o_ref):
    q = q_ref[0]
    k = k_ref[0]
    v = v_ref[0]
    mult = mult_ref[...]
    s = lax.dot_general(q, k, (((1,), (1,)), ((), ())),
                        preferred_element_type=jnp.float32)
    e = mult * jnp.exp2(jnp.minimum(s, 100.0))
    o = lax.dot_general(e, v, (((1,), (0,)), ((), ())),
                        preferred_element_type=jnp.float32)
    o_ref[0] = o[:, :DHEAD] / o[:, DHEAD:DHEAD + 1]


def _attention_tc(qcat, kcat, vaug, mult, interpret=False):
    grid = (MROWS // BM, RHEADS)
    return pl.pallas_call(
        _tc_attn_body,
        grid=grid,
        in_specs=[
            pl.BlockSpec((1, BM, DCAT), lambda i, r: (r, i, 0)),
            pl.BlockSpec((1, NCOLS, DCAT), lambda i, r: (r, 0, 0)),
            pl.BlockSpec((1, NCOLS, DV), lambda i, r: (r, 0, 0)),
            pl.BlockSpec((BM, NCOLS), lambda i, r: (i, 0)),
        ],
        out_specs=pl.BlockSpec((1, BM, DHEAD), lambda i, r: (r, i, 0)),
        out_shape=jax.ShapeDtypeStruct((RHEADS, MROWS, DHEAD), jnp.float32),
        interpret=interpret,
    )(qcat, kcat, vaug, mult)


def _split_cat(x_hi_hi_lo):
    x, order = x_hi_hi_lo
    hi = x.astype(jnp.bfloat16)
    lo = (x - hi.astype(jnp.float32)).astype(jnp.bfloat16)
    parts = {"hhl": (hi, hi, lo), "hlh": (hi, lo, hi)}[order]
    return jnp.concatenate(parts, axis=-1)


def kernel(q3d, k3d, v3d, values, row_indices, row_offsets, column_indices):
    mult = _build_mult(column_indices)
    qcat = _split_cat((q3d * LOG2E, "hhl"))
    kcat = _split_cat((k3d, "hlh"))
    ones = jnp.ones((RHEADS, NCOLS, 1), jnp.float32)
    vaug = jnp.concatenate([v3d, ones], axis=-1).astype(jnp.bfloat16)
    return _attention_tc(qcat, kcat, vaug, mult)


# speedup vs baseline: 20.2716x; 1.0634x over previous
import functools

import jax
import jax.numpy as jnp
from jax import lax
from jax.experimental import pallas as pl
from jax.experimental.pallas import tpu as pltpu
from jax.experimental.pallas import tpu_sc as plsc

RHEADS = 12
MROWS = 2048
NCOLS = 2048
DHEAD = 64
PER_ROW = 128

LOG2E = 1.4426950408889634
DCAT = 3 * DHEAD
DV = DHEAD + 1

NCORES = 2
NSUBCORES = 16
NWORKERS = NCORES * NSUBCORES
ROWS_PER_W = MROWS // NWORKERS
CHUNK_ROWS = 32
NCHUNKS = ROWS_PER_W // CHUNK_ROWS
VECS_PER_ROW = PER_ROW // 16


def _sc_build_mult_body(cols_hbm, zeros_hbm, mult_hbm, idx_v, buf_v):
    wid = lax.axis_index("s") * NCORES + lax.axis_index("c")
    ones16 = jnp.ones((16,), jnp.float32)
    for ci in range(NCHUNKS):
        r0 = wid * ROWS_PER_W + ci * CHUNK_ROWS
        pltpu.sync_copy(cols_hbm.at[pl.ds(r0 * PER_ROW, CHUNK_ROWS * PER_ROW)],
                        idx_v)
        pltpu.sync_copy(zeros_hbm, buf_v)

        def vec_body(i, carry):
            idx = idx_v[pl.ds(i * 16, 16)]
            flat = idx + (i // VECS_PER_ROW) * NCOLS
            plsc.addupdate_scatter(buf_v, [flat], ones16)
            return carry

        lax.fori_loop(0, CHUNK_ROWS * VECS_PER_ROW, vec_body, 0)
        pltpu.sync_copy(buf_v,
                        mult_hbm.at[pl.ds(r0 * NCOLS, CHUNK_ROWS * NCOLS)])


def _build_mult(column_indices):
    mesh = plsc.VectorSubcoreMesh(core_axis_name="c", subcore_axis_name="s")
    zeros = jnp.zeros((CHUNK_ROWS * NCOLS,), jnp.float32)
    fn = functools.partial(
        pl.kernel,
        out_type=jax.ShapeDtypeStruct((MROWS * NCOLS,), jnp.float32),
        mesh=mesh,
        scratch_types=[
            pltpu.VMEM((CHUNK_ROWS * PER_ROW,), jnp.int32),
            pltpu.VMEM((CHUNK_ROWS * NCOLS,), jnp.float32),
        ],
        compiler_params=pltpu.CompilerParams(needs_layout_passes=False),
    )(_sc_build_mult_body)
    return fn(column_indices, zeros).reshape(MROWS, NCOLS)


BM = 256


def _tc_attn_body(q_ref, k_ref, v_ref, mult_ref, o_ref):
    q = q_ref[0]
    k = k_ref[0]
    v = v_ref[0]
    mult = mult_ref[...]
    s = lax.dot_general(q, k, (((1,), (1,)), ((), ())),
                        preferred_element_type=jnp.float32)
    e = mult * jnp.exp2(jnp.minimum(s, 100.0))
    o = lax.dot_general(e, v, (((1,), (0,)), ((), ())),
                        preferred_element_type=jnp.float32)
    o_ref[0] = o[:, :DHEAD] / o[:, DHEAD:DHEAD + 1]


def _attention_tc(qcat, kcat, vaug, mult, interpret=False):
    grid = (MROWS // BM, RHEADS)
    return pl.pallas_call(
        _tc_attn_body,
        grid=grid,
        in_specs=[
            pl.BlockSpec((1, BM, DCAT), lambda i, r: (r, i, 0)),
            pl.BlockSpec((1, NCOLS, DCAT), lambda i, r: (r, 0, 0)),
            pl.BlockSpec((1, NCOLS, DV), lambda i, r: (r, 0, 0)),
            pl.BlockSpec((BM, NCOLS), lambda i, r: (i, 0)),
        ],
        out_specs=pl.BlockSpec((1, BM, DHEAD), lambda i, r: (r, i, 0)),
        out_shape=jax.ShapeDtypeStruct((RHEADS, MROWS, DHEAD), jnp.float32),
        interpret=interpret,
    )(qcat, kcat, vaug, mult)


def _tc_prep_body(q_ref, k_ref, v_ref, qc_ref, kc_ref, va_ref):
    q = q_ref[0] * LOG2E
    qh = q.astype(jnp.bfloat16)
    ql = (q - qh.astype(jnp.float32)).astype(jnp.bfloat16)
    qc_ref[0] = jnp.concatenate([qh, qh, ql], axis=1)
    k = k_ref[0]
    kh = k.astype(jnp.bfloat16)
    kl = (k - kh.astype(jnp.float32)).astype(jnp.bfloat16)
    kc_ref[0] = jnp.concatenate([kh, kl, kh], axis=1)
    ones = jnp.ones((NCOLS, 1), jnp.float32)
    va_ref[0] = jnp.concatenate([v_ref[0], ones], axis=1).astype(jnp.bfloat16)


def _prep_tc(q3d, k3d, v3d, interpret=False):
    return pl.pallas_call(
        _tc_prep_body,
        grid=(RHEADS,),
        in_specs=[
            pl.BlockSpec((1, MROWS, DHEAD), lambda r: (r, 0, 0)),
            pl.BlockSpec((1, NCOLS, DHEAD), lambda r: (r, 0, 0)),
            pl.BlockSpec((1, NCOLS, DHEAD), lambda r: (r, 0, 0)),
        ],
        out_specs=[
            pl.BlockSpec((1, MROWS, DCAT), lambda r: (r, 0, 0)),
            pl.BlockSpec((1, NCOLS, DCAT), lambda r: (r, 0, 0)),
            pl.BlockSpec((1, NCOLS, DV), lambda r: (r, 0, 0)),
        ],
        out_shape=[
            jax.ShapeDtypeStruct((RHEADS, MROWS, DCAT), jnp.bfloat16),
            jax.ShapeDtypeStruct((RHEADS, NCOLS, DCAT), jnp.bfloat16),
            jax.ShapeDtypeStruct((RHEADS, NCOLS, DV), jnp.bfloat16),
        ],
        interpret=interpret,
    )(q3d, k3d, v3d)


def kernel(q3d, k3d, v3d, values, row_indices, row_offsets, column_indices):
    mult = _build_mult(column_indices)
    qcat, kcat, vaug = _prep_tc(q3d, k3d, v3d)
    return _attention_tc(qcat, kcat, vaug, mult)


# speedup vs baseline: 27.5446x; 1.3588x over previous
import functools

import jax
import jax.numpy as jnp
from jax import lax
from jax.experimental import pallas as pl
from jax.experimental.pallas import tpu as pltpu
from jax.experimental.pallas import tpu_sc as plsc

RHEADS = 12
MROWS = 2048
NCOLS = 2048
DHEAD = 64
PER_ROW = 128

LOG2E = 1.4426950408889634
DCAT = 3 * DHEAD
DV = DHEAD + 1

NCORES = 2
NSUBCORES = 16
NWORKERS = NCORES * NSUBCORES
ROWS_PER_W = MROWS // NWORKERS
CHUNK_ROWS = 32
NCHUNKS = ROWS_PER_W // CHUNK_ROWS
VECS_PER_ROW = PER_ROW // 16


def _sc_build_mult_body(cols_hbm, zeros_hbm, mult_hbm, idx_v, buf_v):
    wid = lax.axis_index("s") * NCORES + lax.axis_index("c")
    ones16 = jnp.ones((16,), jnp.float32)
    for ci in range(NCHUNKS):
        r0 = wid * ROWS_PER_W + ci * CHUNK_ROWS
        pltpu.sync_copy(cols_hbm.at[pl.ds(r0 * PER_ROW, CHUNK_ROWS * PER_ROW)],
                        idx_v)
        pltpu.sync_copy(zeros_hbm, buf_v)

        def vec_body(i, carry):
            idx = idx_v[pl.ds(i * 16, 16)]
            flat = idx + (i // VECS_PER_ROW) * NCOLS
            plsc.addupdate_scatter(buf_v, [flat], ones16)
            return carry

        lax.fori_loop(0, CHUNK_ROWS * VECS_PER_ROW, vec_body, 0)
        pltpu.sync_copy(buf_v,
                        mult_hbm.at[pl.ds(r0 * NCOLS, CHUNK_ROWS * NCOLS)])


def _build_mult(column_indices):
    mesh = plsc.VectorSubcoreMesh(core_axis_name="c", subcore_axis_name="s")
    zeros = jnp.zeros((CHUNK_ROWS * NCOLS,), jnp.float32)
    fn = functools.partial(
        pl.kernel,
        out_type=jax.ShapeDtypeStruct((MROWS * NCOLS,), jnp.float32),
        mesh=mesh,
        scratch_types=[
            pltpu.VMEM((CHUNK_ROWS * PER_ROW,), jnp.int32),
            pltpu.VMEM((CHUNK_ROWS * NCOLS,), jnp.float32),
        ],
        compiler_params=pltpu.CompilerParams(needs_layout_passes=False),
    )(_sc_build_mult_body)
    return fn(column_indices, zeros).reshape(MROWS, NCOLS)


BM = 256


def _tc_attn_body(q_ref, k_ref, v_ref, mult_ref, o_ref):
    mult = mult_ref[...]
    for r in range(RHEADS):
        q = q_ref[r]
        k = k_ref[r]
        v = v_ref[r]
        s = lax.dot_general(q, k, (((1,), (1,)), ((), ())),
                            preferred_element_type=jnp.float32)
        e = mult * jnp.exp2(jnp.minimum(s, 100.0))
        o = lax.dot_general(e, v, (((1,), (0,)), ((), ())),
                            preferred_element_type=jnp.float32)
        o_ref[r] = o[:, :DHEAD] / o[:, DHEAD:DHEAD + 1]


def _attention_tc(qcat, kcat, vaug, mult, interpret=False):
    grid = (MROWS // BM,)
    return pl.pallas_call(
        _tc_attn_body,
        grid=grid,
        in_specs=[
            pl.BlockSpec((RHEADS, BM, DCAT), lambda i: (0, i, 0)),
            pl.BlockSpec((RHEADS, NCOLS, DCAT), lambda i: (0, 0, 0)),
            pl.BlockSpec((RHEADS, NCOLS, DV), lambda i: (0, 0, 0)),
            pl.BlockSpec((BM, NCOLS), lambda i: (i, 0)),
        ],
        out_specs=pl.BlockSpec((RHEADS, BM, DHEAD), lambda i: (0, i, 0)),
        out_shape=jax.ShapeDtypeStruct((RHEADS, MROWS, DHEAD), jnp.float32),
        interpret=interpret,
    )(qcat, kcat, vaug, mult)


def _tc_prep_body(q_ref, k_ref, v_ref, qc_ref, kc_ref, va_ref):
    q = q_ref[0] * LOG2E
    qh = q.astype(jnp.bfloat16)
    ql = (q - qh.astype(jnp.float32)).astype(jnp.bfloat16)
    qc_ref[0] = jnp.concatenate([qh, qh, ql], axis=1)
    k = k_ref[0]
    kh = k.astype(jnp.bfloat16)
    kl = (k - kh.astype(jnp.float32)).astype(jnp.bfloat16)
    kc_ref[0] = jnp.concatenate([kh, kl, kh], axis=1)
    ones = jnp.ones((NCOLS, 1), jnp.float32)
    va_ref[0] = jnp.concatenate([v_ref[0], ones], axis=1).astype(jnp.bfloat16)


def _prep_tc(q3d, k3d, v3d, interpret=False):
    return pl.pallas_call(
        _tc_prep_body,
        grid=(RHEADS,),
        in_specs=[
            pl.BlockSpec((1, MROWS, DHEAD), lambda r: (r, 0, 0)),
            pl.BlockSpec((1, NCOLS, DHEAD), lambda r: (r, 0, 0)),
            pl.BlockSpec((1, NCOLS, DHEAD), lambda r: (r, 0, 0)),
        ],
        out_specs=[
            pl.BlockSpec((1, MROWS, DCAT), lambda r: (r, 0, 0)),
            pl.BlockSpec((1, NCOLS, DCAT), lambda r: (r, 0, 0)),
            pl.BlockSpec((1, NCOLS, DV), lambda r: (r, 0, 0)),
        ],
        out_shape=[
            jax.ShapeDtypeStruct((RHEADS, MROWS, DCAT), jnp.bfloat16),
            jax.ShapeDtypeStruct((RHEADS, NCOLS, DCAT), jnp.bfloat16),
            jax.ShapeDtypeStruct((RHEADS, NCOLS, DV), jnp.bfloat16),
        ],
        interpret=interpret,
    )(q3d, k3d, v3d)


def kernel(q3d, k3d, v3d, values, row_indices, row_offsets, column_indices):
    mult = _build_mult(column_indices)
    qcat, kcat, vaug = _prep_tc(q3d, k3d, v3d)
    return _attention_tc(qcat, kcat, vaug, mult)


# speedup vs baseline: 29.2775x; 1.0629x over previous
import functools

import jax
import jax.numpy as jnp
from jax import lax
from jax.experimental import pallas as pl
from jax.experimental.pallas import tpu as pltpu
from jax.experimental.pallas import tpu_sc as plsc

RHEADS = 12
MROWS = 2048
NCOLS = 2048
DHEAD = 64
PER_ROW = 128

LOG2E = 1.4426950408889634
DCAT = 3 * DHEAD
DV = DHEAD + 1

NCORES = 2
NSUBCORES = 16
NWORKERS = NCORES * NSUBCORES
ROWS_PER_W = MROWS // NWORKERS
CHUNK_ROWS = 16
NCHUNKS = ROWS_PER_W // CHUNK_ROWS
VECS_PER_ROW = PER_ROW // 16
CHUNK_WORDS = CHUNK_ROWS * NCOLS


def _sc_build_mult_body(cols_hbm, mult_hbm,
                        idx0, idx1, buf0, buf1, si0, si1, sw0, sw1):
    wid = lax.axis_index("s") * NCORES + lax.axis_index("c")
    base = wid * ROWS_PER_W
    idx_v = [idx0, idx1]
    buf_v = [buf0, buf1]
    sem_i = [si0, si1]
    sem_w = [sw0, sw1]
    ones16 = jnp.ones((16,), jnp.float32)
    zeros16 = jnp.zeros((16,), jnp.float32)

    idx_cp = [None, None]
    wb_cp = [None, None]
    idx_cp[0] = pltpu.async_copy(
        cols_hbm.at[pl.ds(base * PER_ROW, CHUNK_ROWS * PER_ROW)],
        idx_v[0], sem_i[0])
    for ci in range(NCHUNKS):
        b = ci % 2
        r0 = base + ci * CHUNK_ROWS
        if ci + 1 < NCHUNKS:
            idx_cp[1 - b] = pltpu.async_copy(
                cols_hbm.at[pl.ds((r0 + CHUNK_ROWS) * PER_ROW,
                                  CHUNK_ROWS * PER_ROW)],
                idx_v[1 - b], sem_i[1 - b])
        if wb_cp[b] is not None:
            wb_cp[b].wait()

        def zero_body(i, carry, _buf=buf_v[b]):
            for j in range(8):
                _buf[pl.ds(i * 128 + j * 16, 16)] = zeros16
            return carry

        lax.fori_loop(0, CHUNK_WORDS // 128, zero_body, 0)
        idx_cp[b].wait()

        def vec_body(i, carry, _buf=buf_v[b], _idx=idx_v[b]):
            idx = _idx[pl.ds(i * 16, 16)]
            flat = idx + (i // VECS_PER_ROW) * NCOLS
            plsc.addupdate_scatter(_buf, [flat], ones16)
            return carry

        lax.fori_loop(0, CHUNK_ROWS * VECS_PER_ROW, vec_body, 0)
        wb_cp[b] = pltpu.async_copy(
            buf_v[b], mult_hbm.at[pl.ds(r0 * NCOLS, CHUNK_WORDS)], sem_w[b])
    wb_cp[0].wait()
    wb_cp[1].wait()


def _build_mult(column_indices):
    mesh = plsc.VectorSubcoreMesh(core_axis_name="c", subcore_axis_name="s")
    fn = functools.partial(
        pl.kernel,
        out_type=jax.ShapeDtypeStruct((MROWS * NCOLS,), jnp.float32),
        mesh=mesh,
        scratch_types=[
            pltpu.VMEM((CHUNK_ROWS * PER_ROW,), jnp.int32),
            pltpu.VMEM((CHUNK_ROWS * PER_ROW,), jnp.int32),
            pltpu.VMEM((CHUNK_WORDS,), jnp.float32),
            pltpu.VMEM((CHUNK_WORDS,), jnp.float32),
            pltpu.SemaphoreType.DMA,
            pltpu.SemaphoreType.DMA,
            pltpu.SemaphoreType.DMA,
            pltpu.SemaphoreType.DMA,
        ],
        compiler_params=pltpu.CompilerParams(needs_layout_passes=False),
    )(_sc_build_mult_body)
    return fn(column_indices).reshape(MROWS, NCOLS)


BM = 256


def _tc_attn_body(q_ref, k_ref, v_ref, mult_ref, o_ref):
    mult = mult_ref[...]
    for r in range(RHEADS):
        q = q_ref[r]
        k = k_ref[r]
        v = v_ref[r]
        s = lax.dot_general(q, k, (((1,), (1,)), ((), ())),
                            preferred_element_type=jnp.float32)
        e = mult * jnp.exp2(jnp.minimum(s, 100.0))
        o = lax.dot_general(e, v, (((1,), (0,)), ((), ())),
                            preferred_element_type=jnp.float32)
        o_ref[r] = o[:, :DHEAD] / o[:, DHEAD:DHEAD + 1]


def _attention_tc(qcat, kcat, vaug, mult, interpret=False):
    grid = (MROWS // BM,)
    return pl.pallas_call(
        _tc_attn_body,
        grid=grid,
        in_specs=[
            pl.BlockSpec((RHEADS, BM, DCAT), lambda i: (0, i, 0)),
            pl.BlockSpec((RHEADS, NCOLS, DCAT), lambda i: (0, 0, 0)),
            pl.BlockSpec((RHEADS, NCOLS, DV), lambda i: (0, 0, 0)),
            pl.BlockSpec((BM, NCOLS), lambda i: (i, 0)),
        ],
        out_specs=pl.BlockSpec((RHEADS, BM, DHEAD), lambda i: (0, i, 0)),
        out_shape=jax.ShapeDtypeStruct((RHEADS, MROWS, DHEAD), jnp.float32),
        interpret=interpret,
    )(qcat, kcat, vaug, mult)


def _tc_prep_body(q_ref, k_ref, v_ref, qc_ref, kc_ref, va_ref):
    q = q_ref[0] * LOG2E
    qh = q.astype(jnp.bfloat16)
    ql = (q - qh.astype(jnp.float32)).astype(jnp.bfloat16)
    qc_ref[0] = jnp.concatenate([qh, qh, ql], axis=1)
    k = k_ref[0]
    kh = k.astype(jnp.bfloat16)
    kl = (k - kh.astype(jnp.float32)).astype(jnp.bfloat16)
    kc_ref[0] = jnp.concatenate([kh, kl, kh], axis=1)
    ones = jnp.ones((NCOLS, 1), jnp.float32)
    va_ref[0] = jnp.concatenate([v_ref[0], ones], axis=1).astype(jnp.bfloat16)


def _prep_tc(q3d, k3d, v3d, interpret=False):
    return pl.pallas_call(
        _tc_prep_body,
        grid=(RHEADS,),
        in_specs=[
            pl.BlockSpec((1, MROWS, DHEAD), lambda r: (r, 0, 0)),
            pl.BlockSpec((1, NCOLS, DHEAD), lambda r: (r, 0, 0)),
            pl.BlockSpec((1, NCOLS, DHEAD), lambda r: (r, 0, 0)),
        ],
        out_specs=[
            pl.BlockSpec((1, MROWS, DCAT), lambda r: (r, 0, 0)),
            pl.BlockSpec((1, NCOLS, DCAT), lambda r: (r, 0, 0)),
            pl.BlockSpec((1, NCOLS, DV), lambda r: (r, 0, 0)),
        ],
        out_shape=[
            jax.ShapeDtypeStruct((RHEADS, MROWS, DCAT), jnp.bfloat16),
            jax.ShapeDtypeStruct((RHEADS, NCOLS, DCAT), jnp.bfloat16),
            jax.ShapeDtypeStruct((RHEADS, NCOLS, DV), jnp.bfloat16),
        ],
        interpret=interpret,
    )(q3d, k3d, v3d)


def kernel(q3d, k3d, v3d, values, row_indices, row_offsets, column_indices):
    mult = _build_mult(column_indices)
    qcat, kcat, vaug = _prep_tc(q3d, k3d, v3d)
    return _attention_tc(qcat, kcat, vaug, mult)


# speedup vs baseline: 33.0989x; 1.1305x over previous
import functools

import jax
import jax.numpy as jnp
from jax import lax
from jax.experimental import pallas as pl
from jax.experimental.pallas import tpu as pltpu
from jax.experimental.pallas import tpu_sc as plsc

RHEADS = 12
MROWS = 2048
NCOLS = 2048
DHEAD = 64
PER_ROW = 128

LOG2E = 1.4426950408889634
DCAT = 3 * DHEAD
DV = DHEAD + 1

NCORES = 2
NSUBCORES = 16
NWORKERS = NCORES * NSUBCORES
ROWS_PER_W = MROWS // NWORKERS
CHUNK_ROWS = 16
NCHUNKS = ROWS_PER_W // CHUNK_ROWS
VECS_PER_ROW = PER_ROW // 16
CHUNK_WORDS = CHUNK_ROWS * NCOLS


def _sc_build_mult_body(cols_hbm, mult_hbm,
                        idx0, idx1, buf0, buf1, si0, si1, sw0, sw1):
    wid = lax.axis_index("s") * NCORES + lax.axis_index("c")
    base = wid * ROWS_PER_W
    idx_v = [idx0, idx1]
    buf_v = [buf0, buf1]
    sem_i = [si0, si1]
    sem_w = [sw0, sw1]
    ones16 = jnp.ones((16,), jnp.float32)
    zeros16 = jnp.zeros((16,), jnp.float32)

    idx_cp = [None, None]
    wb_cp = [None, None]
    idx_cp[0] = pltpu.async_copy(
        cols_hbm.at[pl.ds(base * PER_ROW, CHUNK_ROWS * PER_ROW)],
        idx_v[0], sem_i[0])
    for ci in range(NCHUNKS):
        b = ci % 2
        r0 = base + ci * CHUNK_ROWS
        if ci + 1 < NCHUNKS:
            idx_cp[1 - b] = pltpu.async_copy(
                cols_hbm.at[pl.ds((r0 + CHUNK_ROWS) * PER_ROW,
                                  CHUNK_ROWS * PER_ROW)],
                idx_v[1 - b], sem_i[1 - b])
        if wb_cp[b] is not None:
            wb_cp[b].wait()

        def zero_body(i, carry, _buf=buf_v[b]):
            for j in range(8):
                _buf[pl.ds(i * 128 + j * 16, 16)] = zeros16
            return carry

        lax.fori_loop(0, CHUNK_WORDS // 128, zero_body, 0)
        idx_cp[b].wait()

        def vec_body(i, carry, _buf=buf_v[b], _idx=idx_v[b]):
            idx = _idx[pl.ds(i * 16, 16)]
            flat = idx + (i // VECS_PER_ROW) * NCOLS
            plsc.addupdate_scatter(_buf, [flat], ones16)
            return carry

        lax.fori_loop(0, CHUNK_ROWS * VECS_PER_ROW, vec_body, 0)
        wb_cp[b] = pltpu.async_copy(
            buf_v[b], mult_hbm.at[pl.ds(r0 * NCOLS, CHUNK_WORDS)], sem_w[b])
    wb_cp[0].wait()
    wb_cp[1].wait()


def _build_mult(column_indices):
    mesh = plsc.VectorSubcoreMesh(core_axis_name="c", subcore_axis_name="s")
    fn = functools.partial(
        pl.kernel,
        out_type=jax.ShapeDtypeStruct((MROWS * NCOLS,), jnp.float32),
        mesh=mesh,
        scratch_types=[
            pltpu.VMEM((CHUNK_ROWS * PER_ROW,), jnp.int32),
            pltpu.VMEM((CHUNK_ROWS * PER_ROW,), jnp.int32),
            pltpu.VMEM((CHUNK_WORDS,), jnp.float32),
            pltpu.VMEM((CHUNK_WORDS,), jnp.float32),
            pltpu.SemaphoreType.DMA,
            pltpu.SemaphoreType.DMA,
            pltpu.SemaphoreType.DMA,
            pltpu.SemaphoreType.DMA,
        ],
        compiler_params=pltpu.CompilerParams(needs_layout_passes=False),
    )(_sc_build_mult_body)
    return fn(column_indices).reshape(MROWS, NCOLS)


BM = 256


def _tc_attn_body(q_ref, k_ref, v_ref, mult_ref, o_ref, kc_ref, va_ref):
    @pl.when(pl.program_id(0) == 0)
    def _prep():
        ones = jnp.ones((NCOLS, 1), jnp.float32)
        for r in range(RHEADS):
            k = k_ref[r]
            kh = k.astype(jnp.bfloat16)
            kl = (k - kh.astype(jnp.float32)).astype(jnp.bfloat16)
            kc_ref[r] = jnp.concatenate([kh, kl, kh], axis=1)
            va_ref[r] = jnp.concatenate(
                [v_ref[r], ones], axis=1).astype(jnp.bfloat16)

    mult = mult_ref[...]
    for r in range(RHEADS):
        qf = q_ref[r] * LOG2E
        qh = qf.astype(jnp.bfloat16)
        ql = (qf - qh.astype(jnp.float32)).astype(jnp.bfloat16)
        q = jnp.concatenate([qh, qh, ql], axis=1)
        s = lax.dot_general(q, kc_ref[r], (((1,), (1,)), ((), ())),
                            preferred_element_type=jnp.float32)
        e = mult * jnp.exp2(jnp.minimum(s, 100.0))
        o = lax.dot_general(e, va_ref[r], (((1,), (0,)), ((), ())),
                            preferred_element_type=jnp.float32)
        o_ref[r] = o[:, :DHEAD] / o[:, DHEAD:DHEAD + 1]


def _attention_tc(q3d, k3d, v3d, mult, interpret=False):
    grid = (MROWS // BM,)
    return pl.pallas_call(
        _tc_attn_body,
        grid=grid,
        in_specs=[
            pl.BlockSpec((RHEADS, BM, DHEAD), lambda i: (0, i, 0)),
            pl.BlockSpec((RHEADS, NCOLS, DHEAD), lambda i: (0, 0, 0)),
            pl.BlockSpec((RHEADS, NCOLS, DHEAD), lambda i: (0, 0, 0)),
            pl.BlockSpec((BM, NCOLS), lambda i: (i, 0)),
        ],
        out_specs=pl.BlockSpec((RHEADS, BM, DHEAD), lambda i: (0, i, 0)),
        out_shape=jax.ShapeDtypeStruct((RHEADS, MROWS, DHEAD), jnp.float32),
        scratch_shapes=[
            pltpu.VMEM((RHEADS, NCOLS, DCAT), jnp.bfloat16),
            pltpu.VMEM((RHEADS, NCOLS, DV), jnp.bfloat16),
        ],
        interpret=interpret,
    )(q3d, k3d, v3d, mult)


def kernel(q3d, k3d, v3d, values, row_indices, row_offsets, column_indices):
    mult = _build_mult(column_indices)
    return _attention_tc(q3d, k3d, v3d, mult)


# speedup vs baseline: 37.9265x; 1.1459x over previous
import functools

import jax
import jax.numpy as jnp
from jax import lax
from jax.experimental import pallas as pl
from jax.experimental.pallas import tpu as pltpu
from jax.experimental.pallas import tpu_sc as plsc

RHEADS = 12
MROWS = 2048
NCOLS = 2048
DHEAD = 64
PER_ROW = 128

LOG2E = 1.4426950408889634
DCAT = 3 * DHEAD
DV = DHEAD + 1

NCORES = 2
NSUBCORES = 16
NWORKERS = NCORES * NSUBCORES
ROWS_PER_W = MROWS // NWORKERS
CHUNK_ROWS = 16
NCHUNKS = ROWS_PER_W // CHUNK_ROWS
VECS_PER_ROW = PER_ROW // 16
CHUNK_WORDS = CHUNK_ROWS * NCOLS


def _sc_build_mult_body(cols_hbm, mult_hbm,
                        idx0, idx1, buf0, buf1, si0, si1, sw0, sw1):
    wid = lax.axis_index("s") * NCORES + lax.axis_index("c")
    base = wid * ROWS_PER_W
    idx_v = [idx0, idx1]
    buf_v = [buf0, buf1]
    sem_i = [si0, si1]
    sem_w = [sw0, sw1]
    ones16 = jnp.ones((16,), jnp.float32)
    zeros16 = jnp.zeros((16,), jnp.float32)

    idx_cp = [None, None]
    wb_cp = [None, None]
    idx_cp[0] = pltpu.async_copy(
        cols_hbm.at[pl.ds(base * PER_ROW, CHUNK_ROWS * PER_ROW)],
        idx_v[0], sem_i[0])
    for ci in range(NCHUNKS):
        b = ci % 2
        r0 = base + ci * CHUNK_ROWS
        if ci + 1 < NCHUNKS:
            idx_cp[1 - b] = pltpu.async_copy(
                cols_hbm.at[pl.ds((r0 + CHUNK_ROWS) * PER_ROW,
                                  CHUNK_ROWS * PER_ROW)],
                idx_v[1 - b], sem_i[1 - b])
        if wb_cp[b] is not None:
            for cp in wb_cp[b]:
                cp.wait()
            wb_cp[b] = None

        def zero_body(i, carry, _buf=buf_v[b]):
            for j in range(8):
                _buf[pl.ds(i * 128 + j * 16, 16)] = zeros16
            return carry

        lax.fori_loop(0, CHUNK_WORDS // 128, zero_body, 0)
        idx_cp[b].wait()

        def vec_body(i, carry, _buf=buf_v[b], _idx=idx_v[b]):
            idx = _idx[pl.ds(i * 16, 16)]
            flat = idx + (i // VECS_PER_ROW) * NCOLS
            plsc.addupdate_scatter(_buf, [flat], ones16)
            return carry

        lax.fori_loop(0, CHUNK_ROWS * VECS_PER_ROW, vec_body, 0)
        wb_cp[b] = [
            pltpu.async_copy(buf_v[b].at[pl.ds(row * NCOLS, NCOLS)],
                             mult_hbm.at[r0 + row], sem_w[b])
            for row in range(CHUNK_ROWS)
        ]
    for cps in wb_cp:
        if cps is not None:
            for cp in cps:
                cp.wait()


def _build_mult(column_indices):
    mesh = plsc.VectorSubcoreMesh(core_axis_name="c", subcore_axis_name="s")
    fn = functools.partial(
        pl.kernel,
        out_type=jax.ShapeDtypeStruct((MROWS, NCOLS), jnp.float32),
        mesh=mesh,
        scratch_types=[
            pltpu.VMEM((CHUNK_ROWS * PER_ROW,), jnp.int32),
            pltpu.VMEM((CHUNK_ROWS * PER_ROW,), jnp.int32),
            pltpu.VMEM((CHUNK_WORDS,), jnp.float32),
            pltpu.VMEM((CHUNK_WORDS,), jnp.float32),
            pltpu.SemaphoreType.DMA,
            pltpu.SemaphoreType.DMA,
            pltpu.SemaphoreType.DMA,
            pltpu.SemaphoreType.DMA,
        ],
        compiler_params=pltpu.CompilerParams(needs_layout_passes=False),
    )(_sc_build_mult_body)
    return fn(column_indices)


BM = 256


def _tc_attn_body(q_ref, k_ref, v_ref, mult_ref, o_ref, kc_ref, va_ref):
    @pl.when(pl.program_id(0) == 0)
    def _prep():
        ones = jnp.ones((NCOLS, 1), jnp.float32)
        for r in range(RHEADS):
            k = k_ref[r]
            kh = k.astype(jnp.bfloat16)
            kl = (k - kh.astype(jnp.float32)).astype(jnp.bfloat16)
            kc_ref[r] = jnp.concatenate([kh, kl, kh], axis=1)
            va_ref[r] = jnp.concatenate(
                [v_ref[r], ones], axis=1).astype(jnp.bfloat16)

    mult = mult_ref[...]
    for r in range(RHEADS):
        qf = q_ref[r] * LOG2E
        qh = qf.astype(jnp.bfloat16)
        ql = (qf - qh.astype(jnp.float32)).astype(jnp.bfloat16)
        q = jnp.concatenate([qh, qh, ql], axis=1)
        s = lax.dot_general(q, kc_ref[r], (((1,), (1,)), ((), ())),
                            preferred_element_type=jnp.float32)
        e = mult * jnp.exp2(jnp.minimum(s, 100.0))
        o = lax.dot_general(e, va_ref[r], (((1,), (0,)), ((), ())),
                            preferred_element_type=jnp.float32)
        o_ref[r] = o[:, :DHEAD] / o[:, DHEAD:DHEAD + 1]


def _attention_tc(q3d, k3d, v3d, mult, interpret=False):
    grid = (MROWS // BM,)
    return pl.pallas_call(
        _tc_attn_body,
        grid=grid,
        in_specs=[
            pl.BlockSpec((RHEADS, BM, DHEAD), lambda i: (0, i, 0)),
            pl.BlockSpec((RHEADS, NCOLS, DHEAD), lambda i: (0, 0, 0)),
            pl.BlockSpec((RHEADS, NCOLS, DHEAD), lambda i: (0, 0, 0)),
            pl.BlockSpec((BM, NCOLS), lambda i: (i, 0)),
        ],
        out_specs=pl.BlockSpec((RHEADS, BM, DHEAD), lambda i: (0, i, 0)),
        out_shape=jax.ShapeDtypeStruct((RHEADS, MROWS, DHEAD), jnp.float32),
        scratch_shapes=[
            pltpu.VMEM((RHEADS, NCOLS, DCAT), jnp.bfloat16),
            pltpu.VMEM((RHEADS, NCOLS, DV), jnp.bfloat16),
        ],
        interpret=interpret,
    )(q3d, k3d, v3d, mult)


def kernel(q3d, k3d, v3d, values, row_indices, row_offsets, column_indices):
    mult = _build_mult(column_indices)
    return _attention_tc(q3d, k3d, v3d, mult)
